# baseline (speedup 1.0000x reference)
import jax
import jax.numpy as jnp
from jax import lax
from jax.experimental import pallas as pl
from jax.experimental.pallas import tpu as pltpu
from jax.experimental.pallas import tpu_sc as plsc

B = 4096 * 50
D = 32
NC, NS, L = 2, 16, 16
NW = NC * NS
B_PER_W = B // NW
CH = 320
NCH = B_PER_W // CH

_GATHER_DNUMS = lax.GatherDimensionNumbers(
    offset_dims=(), collapsed_slice_dims=(0,), start_index_map=(0,))


def _splat_lane(vec, lane):
    idx = jnp.full((L, 1), lane, jnp.int32)
    return lax.gather(vec, idx, _GATHER_DNUMS, slice_sizes=(1,),
                      mode=lax.GatherScatterMode.PROMISE_IN_BOUNDS)


def _emb_body(x_hbm, table_hbm, out_hbm, idx_v, rows0, rows1, flag_v,
              shared_f, flag_s, gsem0, gsem1, ssem0, ssem1):
    rows = (rows0, rows1)
    gsem = (gsem0, gsem1)
    ssem = (ssem0, ssem1)
    cid = lax.axis_index("c")
    sid = lax.axis_index("s")
    wid = sid * NC + cid

    pltpu.sync_copy(x_hbm.at[wid], idx_v)

    def gather_start(c, b):
        pltpu.async_copy(table_hbm.at[idx_v.at[pl.ds(c * CH, CH)]],
                         rows[b], gsem[b])

    def gather_wait(c, b):
        pltpu.make_async_copy(table_hbm.at[idx_v.at[pl.ds(c * CH, CH)]],
                              rows[b], gsem[b]).wait()

    def store_start(c, b):
        pltpu.async_copy(rows[b].at[:, pl.ds(0, D)],
                         out_hbm.at[wid, pl.ds(c * CH, CH)], ssem[b])

    def store_wait(c, b):
        pltpu.make_async_copy(rows[b].at[:, pl.ds(0, D)],
                              out_hbm.at[wid, pl.ds(c * CH, CH)],
                              ssem[b]).wait()

    gather_start(0, 0)

    def scan_j(j, a):
        return jnp.minimum(a, idx_v[pl.ds(j * L, L)])

    mn = lax.fori_loop(0, B_PER_W // L, scan_j,
                       jnp.full((L,), 2**31 - 1, jnp.int32))

    flag_v[pl.ds(0, L)] = mn
    pltpu.sync_copy(flag_v, shared_f.at[sid])
    pltpu.sync_copy(shared_f.at[sid], flag_s)
    hz = flag_s[0]
    for l in range(1, L):
        hz = jnp.minimum(hz, flag_s[l])
    has_zero = hz == 0

    for c in range(NCH):
        b = c % 2
        gather_wait(c, b)
        if c + 1 < NCH:
            if c >= 1:
                store_wait(c - 1, 1 - b)
            gather_start(c + 1, 1 - b)

        @pl.when(has_zero)
        def _(c=c, b=b):
            def mask16(j, carry):
                v = idx_v[pl.ds(c * CH + j * L, L)]
                mf = jnp.where(v == 0, 0.0, 1.0)
                for l in range(L):
                    mb = _splat_lane(mf, l)
                    r = j * L + l
                    lo = rows[b][r, pl.ds(0, L)]
                    hi = rows[b][r, pl.ds(L, L)]
                    rows[b][r, pl.ds(0, L)] = lo * mb
                    rows[b][r, pl.ds(L, L)] = hi * mb
                return carry

            lax.fori_loop(0, CH // L, mask16, 0)

        store_start(c, b)

    store_wait(NCH - 2, NCH % 2)
    store_wait(NCH - 1, (NCH - 1) % 2)


def _emb(x2d, table):
    mesh = plsc.VectorSubcoreMesh(core_axis_name="c", subcore_axis_name="s")
    return pl.kernel(
        _emb_body,
        out_type=jax.ShapeDtypeStruct((NW, B_PER_W, D), jnp.float32),
        mesh=mesh,
        compiler_params=pltpu.CompilerParams(use_tc_tiling_on_sc=False),
        scratch_types=[
            pltpu.VMEM((B_PER_W,), jnp.int32),
            pltpu.VMEM((CH, 128), jnp.float32),
            pltpu.VMEM((CH, 128), jnp.float32),
            pltpu.VMEM((L,), jnp.int32),
            pltpu.VMEM_SHARED((NS, L), jnp.int32),
            pltpu.SMEM((L,), jnp.int32),
            pltpu.SemaphoreType.DMA,
            pltpu.SemaphoreType.DMA,
            pltpu.SemaphoreType.DMA,
            pltpu.SemaphoreType.DMA,
        ],
    )(x2d, table)


def kernel(x, table):
    x2d = x.reshape(NW, B_PER_W).astype(jnp.int32)
    tp = jnp.pad(table, ((0, 0), (0, 96)))
    out = _emb(x2d, tp)
    return out.reshape(x.shape[0], x.shape[1], D)


# speedup vs baseline: 1.0378x; 1.0378x over previous
import jax
import jax.numpy as jnp
from jax import lax
from jax.experimental import pallas as pl
from jax.experimental.pallas import tpu as pltpu
from jax.experimental.pallas import tpu_sc as plsc

B = 4096 * 50
D = 32
NC, NS, L = 2, 16, 16
NW = NC * NS
B_PER_W = B // NW
CH = 1600
NCH = B_PER_W // CH

_GATHER_DNUMS = lax.GatherDimensionNumbers(
    offset_dims=(), collapsed_slice_dims=(0,), start_index_map=(0,))


def _splat_lane(vec, lane):
    idx = jnp.full((L, 1), lane, jnp.int32)
    return lax.gather(vec, idx, _GATHER_DNUMS, slice_sizes=(1,),
                      mode=lax.GatherScatterMode.PROMISE_IN_BOUNDS)


def _emb_body(x_hbm, table_hbm, out_hbm, idx_v, rows0, rows1, flag_v,
              shared_f, flag_s, gsem0, gsem1, ssem0, ssem1):
    rows = (rows0, rows1)
    gsem = (gsem0, gsem1)
    ssem = (ssem0, ssem1)
    cid = lax.axis_index("c")
    sid = lax.axis_index("s")
    wid = sid * NC + cid

    pltpu.sync_copy(x_hbm.at[wid], idx_v)

    def gather_start(c, b):
        pltpu.async_copy(table_hbm.at[idx_v.at[pl.ds(c * CH, CH)]],
                         rows[b], gsem[b])

    def gather_wait(c, b):
        pltpu.make_async_copy(table_hbm.at[idx_v.at[pl.ds(c * CH, CH)]],
                              rows[b], gsem[b]).wait()

    def store_start(c, b):
        pltpu.async_copy(rows[b], out_hbm.at[wid, pl.ds(c * CH, CH)],
                         ssem[b])

    def store_wait(c, b):
        pltpu.make_async_copy(rows[b], out_hbm.at[wid, pl.ds(c * CH, CH)],
                              ssem[b]).wait()

    gather_start(0, 0)

    def scan_j(j, a):
        return jnp.minimum(a, idx_v[pl.ds(j * L, L)])

    mn = lax.fori_loop(0, B_PER_W // L, scan_j,
                       jnp.full((L,), 2**31 - 1, jnp.int32))

    flag_v[pl.ds(0, L)] = mn
    pltpu.sync_copy(flag_v, shared_f.at[sid])
    pltpu.sync_copy(shared_f.at[sid], flag_s)
    hz = flag_s[0]
    for l in range(1, L):
        hz = jnp.minimum(hz, flag_s[l])
    has_zero = hz == 0

    for c in range(NCH):
        b = c % 2
        gather_wait(c, b)
        if c + 1 < NCH:
            if c >= 1:
                store_wait(c - 1, 1 - b)
            gather_start(c + 1, 1 - b)

        @pl.when(has_zero)
        def _(c=c, b=b):
            def mask16(j, carry):
                v = idx_v[pl.ds(c * CH + j * L, L)]
                mf = jnp.where(v == 0, 0.0, 1.0)
                for l in range(L):
                    mb = _splat_lane(mf, l)
                    r = j * L + l
                    lo = rows[b][r, pl.ds(0, L)]
                    hi = rows[b][r, pl.ds(L, L)]
                    rows[b][r, pl.ds(0, L)] = lo * mb
                    rows[b][r, pl.ds(L, L)] = hi * mb
                return carry

            lax.fori_loop(0, CH // L, mask16, 0)

        store_start(c, b)

    store_wait(NCH - 2, NCH % 2)
    store_wait(NCH - 1, (NCH - 1) % 2)


def _emb(x2d, table):
    mesh = plsc.VectorSubcoreMesh(core_axis_name="c", subcore_axis_name="s")
    return pl.kernel(
        _emb_body,
        out_type=jax.ShapeDtypeStruct((NW, B_PER_W, D), jnp.float32),
        mesh=mesh,
        compiler_params=pltpu.CompilerParams(use_tc_tiling_on_sc=False),
        scratch_types=[
            pltpu.VMEM((B_PER_W,), jnp.int32),
            pltpu.VMEM((CH, D), jnp.float32),
            pltpu.VMEM((CH, D), jnp.float32),
            pltpu.VMEM((L,), jnp.int32),
            pltpu.VMEM_SHARED((NS, L), jnp.int32),
            pltpu.SMEM((L,), jnp.int32),
            pltpu.SemaphoreType.DMA,
            pltpu.SemaphoreType.DMA,
            pltpu.SemaphoreType.DMA,
            pltpu.SemaphoreType.DMA,
        ],
    )(x2d, table)


def kernel(x, table):
    x2d = x.reshape(NW, B_PER_W).astype(jnp.int32)
    t = lax.optimization_barrier(table.reshape(250000, 128))
    out = _emb(x2d, t.reshape(1000000, 32))
    return out.reshape(x.shape[0], x.shape[1], D)
